# Initial kernel scaffold; baseline (speedup 1.0000x reference)
import jax
import jax.numpy as jnp
from jax.experimental import pallas as pl


def kernel(rna_data, tissue_id, tissue_table, seq_table, W1, b1, W2, b2, W3, b3):
    raise NotImplementedError("write your pallas kernel here")


# speedup vs baseline: 4.4530x; 4.4530x over previous
import functools

import jax
import jax.numpy as jnp
from jax import lax
from jax.experimental import pallas as pl
from jax.experimental.pallas import tpu as pltpu
from jax.experimental.pallas import tpu_sc as plsc

MAX_SEQ_LENGTH = 200
MAX_NORM = 2.0
D = 32
SEQ = 150
P = SEQ + 2
TAB = 96
HID1 = 512
HID2 = 256

NC, NS = 2, 16
NW = NC * NS
CH = 128


def _renorm_body(c_ref, o_ref):
    x = c_ref[:]
    n = jnp.sqrt(jnp.sum(x * x, axis=1, keepdims=True))
    scale = jnp.where(n > MAX_NORM, MAX_NORM / (n + 1e-7), 1.0)
    o_ref[:] = x * scale


def _renorm_table(c):
    return pl.pallas_call(
        _renorm_body,
        out_shape=jax.ShapeDtypeStruct(c.shape, jnp.float32),
    )(c)


def _make_gather(rows):
    rpw = rows // NW
    nch = rpw // CH
    mesh = plsc.VectorSubcoreMesh(
        core_axis_name="c", subcore_axis_name="s",
        num_cores=NC, num_subcores=NS)

    @functools.partial(
        pl.kernel,
        mesh=mesh,
        compiler_params=pltpu.CompilerParams(use_tc_tiling_on_sc=False),
        out_type=jax.ShapeDtypeStruct((rows, D), jnp.float32),
        scratch_types=[
            pltpu.VMEM((rpw,), jnp.int32),
            pltpu.VMEM((CH, D), jnp.float32),
            pltpu.SemaphoreType.DMA,
        ],
    )
    def gather(tab_hbm, idx_hbm, out_hbm, idx_v, rows_v, sem):
        wid = lax.axis_index("s") * NC + lax.axis_index("c")
        base = wid * rpw
        pltpu.sync_copy(idx_hbm.at[pl.ds(base, rpw)], idx_v)

        def body(c, carry):
            off = pl.multiple_of(c * CH, CH)
            pltpu.async_copy(
                tab_hbm.at[idx_v.at[pl.ds(off, CH)]], rows_v, sem).wait()
            pltpu.sync_copy(rows_v, out_hbm.at[pl.ds(base + off, CH)])
            return carry

        lax.fori_loop(0, nch, body, 0)

    return gather


def _mlp_body(x_ref, w1_ref, b1_ref, w2_ref, b2_ref, w3_ref, b3_ref, o_ref):
    h1 = jnp.dot(x_ref[:], w1_ref[:], preferred_element_type=jnp.float32)
    h1 = jnp.maximum(h1 + b1_ref[:], 0.0)
    h2 = jnp.dot(h1, w2_ref[:], preferred_element_type=jnp.float32)
    h2 = jnp.maximum(h2 + b2_ref[:], 0.0)
    o_ref[:] = jnp.dot(h2, w3_ref[:], preferred_element_type=jnp.float32) + b3_ref[:]


def _mlp(x, w1, b1, w2, b2, w3, b3, bt):
    b_rows, feat = x.shape
    grid = (b_rows // bt,)
    return pl.pallas_call(
        _mlp_body,
        grid=grid,
        in_specs=[
            pl.BlockSpec((bt, feat), lambda i: (i, 0)),
            pl.BlockSpec(w1.shape, lambda i: (0, 0)),
            pl.BlockSpec(b1.shape, lambda i: (0, 0)),
            pl.BlockSpec(w2.shape, lambda i: (0, 0)),
            pl.BlockSpec(b2.shape, lambda i: (0, 0)),
            pl.BlockSpec(w3.shape, lambda i: (0, 0)),
            pl.BlockSpec(b3.shape, lambda i: (0, 0)),
        ],
        out_specs=pl.BlockSpec((bt, 128), lambda i: (i, 0)),
        out_shape=jax.ShapeDtypeStruct((b_rows, 128), jnp.float32),
    )(x, w1, b1, w2, b2, w3, b3)


def kernel(rna_data, tissue_id, tissue_table, seq_table, W1, b1, W2, b2, W3, b3):
    B, S = rna_data.shape

    tab = jnp.concatenate(
        [seq_table, tissue_table, jnp.zeros((1, D), jnp.float32)], axis=0)
    tab = _renorm_table(tab)

    idx = jnp.concatenate(
        [tissue_id[:, None].astype(jnp.int32) + 65,
         rna_data.astype(jnp.int32),
         jnp.full((B, 1), TAB - 1, jnp.int32)], axis=1)
    idx_flat = idx.reshape(-1)

    rows = B * P
    x = _make_gather(rows)(tab, idx_flat)
    x2 = x.reshape(B, P * D)

    w1e = jnp.concatenate(
        [W1[:D + SEQ * D], jnp.zeros((D, HID1), jnp.float32)], axis=0)
    w3e = jnp.concatenate(
        [W3, jnp.zeros((HID2, 127), jnp.float32)], axis=1)
    b3e = jnp.concatenate([b3, jnp.zeros((127,), jnp.float32)])[None, :]

    out = _mlp(x2, w1e, b1[None, :], W2, b2[None, :], w3e, b3e, bt=512)
    return out[:, :1]


# speedup vs baseline: 4.5618x; 1.0244x over previous
import functools

import jax
import jax.numpy as jnp
from jax import lax
from jax.experimental import pallas as pl
from jax.experimental.pallas import tpu as pltpu
from jax.experimental.pallas import tpu_sc as plsc

MAX_SEQ_LENGTH = 200
MAX_NORM = 2.0
D = 32
SEQ = 150
P = SEQ + 2
TAB = 96
HID1 = 512
HID2 = 256

NC, NS = 2, 16
NW = NC * NS
CH = 128


def _renorm_body(c_ref, o_ref):
    x = c_ref[:]
    n = jnp.sqrt(jnp.sum(x * x, axis=1, keepdims=True))
    scale = jnp.where(n > MAX_NORM, MAX_NORM / (n + 1e-7), 1.0)
    o_ref[:] = x * scale


def _renorm_table(c):
    return pl.pallas_call(
        _renorm_body,
        out_shape=jax.ShapeDtypeStruct(c.shape, jnp.float32),
    )(c)


def _make_gather(rows):
    rpw = rows // NW
    nch = rpw // CH
    K = 4
    HALF = K * CH
    nout = nch // K // 2
    mesh = plsc.VectorSubcoreMesh(
        core_axis_name="c", subcore_axis_name="s",
        num_cores=NC, num_subcores=NS)

    @functools.partial(
        pl.kernel,
        mesh=mesh,
        compiler_params=pltpu.CompilerParams(use_tc_tiling_on_sc=False),
        out_type=jax.ShapeDtypeStruct((rows, D), jnp.float32),
        scratch_types=[
            pltpu.VMEM((rpw,), jnp.int32),
            pltpu.VMEM((2 * HALF, D), jnp.float32),
            pltpu.SemaphoreType.DMA,
            pltpu.SemaphoreType.DMA,
            pltpu.SemaphoreType.DMA,
            pltpu.SemaphoreType.DMA,
        ],
    )
    def gather(tab_hbm, idx_hbm, out_hbm, idx_v, buf, gsem0, gsem1, wsem0, wsem1):
        wid = lax.axis_index("s") * NC + lax.axis_index("c")
        base = wid * rpw
        pltpu.sync_copy(idx_hbm.at[pl.ds(base, rpw)], idx_v)
        gsems = (gsem0, gsem1)
        wsems = (wsem0, wsem1)

        def issue(h, g):
            for b in range(K):
                off = pl.multiple_of(g * HALF + b * CH, CH)
                pltpu.async_copy(
                    tab_hbm.at[idx_v.at[pl.ds(off, CH)]],
                    buf.at[pl.ds(h * HALF + b * CH, CH)],
                    gsems[h])

        def wait_half(h, sem):
            pltpu.make_async_copy(
                out_hbm.at[pl.ds(base, HALF)],
                buf.at[pl.ds(h * HALF, HALF)],
                sem).wait()

        def write(h, g):
            pltpu.async_copy(
                buf.at[pl.ds(h * HALF, HALF)],
                out_hbm.at[pl.ds(base + g * HALF, HALF)],
                wsems[h])

        issue(0, 0)

        def outer(i, carry):
            @pl.when(i > 0)
            def _():
                wait_half(1, wsems[1])
            issue(1, 2 * i + 1)
            wait_half(0, gsems[0])
            write(0, 2 * i)

            @pl.when(i + 1 < nout)
            def _():
                wait_half(0, wsems[0])
                issue(0, 2 * i + 2)

            wait_half(1, gsems[1])
            write(1, 2 * i + 1)
            return carry

        lax.fori_loop(0, nout, outer, 0)
        wait_half(0, wsems[0])
        wait_half(1, wsems[1])

    return gather


def _mlp_body(x_ref, w1_ref, b1_ref, w2_ref, b2_ref, w3_ref, b3_ref, o_ref):
    h1 = jnp.dot(x_ref[:], w1_ref[:], preferred_element_type=jnp.float32)
    h1 = jnp.maximum(h1 + b1_ref[:], 0.0)
    h2 = jnp.dot(h1, w2_ref[:], preferred_element_type=jnp.float32)
    h2 = jnp.maximum(h2 + b2_ref[:], 0.0)
    o_ref[:] = jnp.dot(h2, w3_ref[:], preferred_element_type=jnp.float32) + b3_ref[:]


def _mlp(x, w1, b1, w2, b2, w3, b3, bt):
    b_rows, feat = x.shape
    grid = (b_rows // bt,)
    return pl.pallas_call(
        _mlp_body,
        grid=grid,
        in_specs=[
            pl.BlockSpec((bt, feat), lambda i: (i, 0)),
            pl.BlockSpec(w1.shape, lambda i: (0, 0)),
            pl.BlockSpec(b1.shape, lambda i: (0, 0)),
            pl.BlockSpec(w2.shape, lambda i: (0, 0)),
            pl.BlockSpec(b2.shape, lambda i: (0, 0)),
            pl.BlockSpec(w3.shape, lambda i: (0, 0)),
            pl.BlockSpec(b3.shape, lambda i: (0, 0)),
        ],
        out_specs=pl.BlockSpec((bt, 128), lambda i: (i, 0)),
        out_shape=jax.ShapeDtypeStruct((b_rows, 128), jnp.float32),
    )(x, w1, b1, w2, b2, w3, b3)


def kernel(rna_data, tissue_id, tissue_table, seq_table, W1, b1, W2, b2, W3, b3):
    B, S = rna_data.shape

    tab = jnp.concatenate(
        [seq_table, tissue_table, jnp.zeros((1, D), jnp.float32)], axis=0)
    tab = _renorm_table(tab)

    idx = jnp.concatenate(
        [tissue_id[:, None].astype(jnp.int32) + 65,
         rna_data.astype(jnp.int32),
         jnp.full((B, 1), TAB - 1, jnp.int32)], axis=1)
    idx_flat = idx.reshape(-1)

    rows = B * P
    x = _make_gather(rows)(tab, idx_flat)
    x2 = x.reshape(B, P * D)

    w1e = jnp.concatenate(
        [W1[:D + SEQ * D], jnp.zeros((D, HID1), jnp.float32)], axis=0)
    w3e = jnp.concatenate(
        [W3, jnp.zeros((HID2, 127), jnp.float32)], axis=1)
    b3e = jnp.concatenate([b3, jnp.zeros((127,), jnp.float32)])[None, :]

    out = _mlp(x2, w1e, b1[None, :], W2, b2[None, :], w3e, b3e, bt=512)
    return out[:, :1]


# speedup vs baseline: 5.1450x; 1.1279x over previous
import functools

import jax
import jax.numpy as jnp
from jax import lax
from jax.experimental import pallas as pl
from jax.experimental.pallas import tpu as pltpu
from jax.experimental.pallas import tpu_sc as plsc

MAX_SEQ_LENGTH = 200
MAX_NORM = 2.0
D = 32
SEQ = 150
P = SEQ + 2
TAB = 96
HID1 = 512
HID2 = 256

NC, NS = 2, 16
NW = NC * NS
CH = 128


def _renorm_body(c_ref, o_ref):
    x = c_ref[:]
    n = jnp.sqrt(jnp.sum(x * x, axis=1, keepdims=True))
    scale = jnp.where(n > MAX_NORM, MAX_NORM / (n + 1e-7), 1.0)
    o_ref[:] = x * scale


def _renorm_table(c):
    return pl.pallas_call(
        _renorm_body,
        out_shape=jax.ShapeDtypeStruct(c.shape, jnp.float32),
    )(c)


def _make_gather(rows):
    DW = 16
    GT = 512
    GW = GT * DW
    rpw = rows // NW
    nout = rpw // GT // 2
    mesh = plsc.VectorSubcoreMesh(
        core_axis_name="c", subcore_axis_name="s",
        num_cores=NC, num_subcores=NS)

    @functools.partial(
        pl.kernel,
        mesh=mesh,
        compiler_params=pltpu.CompilerParams(
            use_tc_tiling_on_sc=False, needs_layout_passes=False),
        out_type=jax.ShapeDtypeStruct((rows * DW,), jnp.int32),
        scratch_types=[
            pltpu.VMEM((rpw,), jnp.int32),
            pltpu.VMEM((TAB * DW,), jnp.int32),
            pltpu.VMEM((2 * GW,), jnp.int32),
            pltpu.SemaphoreType.DMA,
            pltpu.SemaphoreType.DMA,
        ],
    )
    def gather(tab_hbm, idx_hbm, out_hbm, idx_v, tab_v, stage, wsem0, wsem1):
        wid = lax.axis_index("s") * NC + lax.axis_index("c")
        base = wid * rpw
        pltpu.sync_copy(tab_hbm, tab_v)
        pltpu.sync_copy(idx_hbm.at[pl.ds(base, rpw)], idx_v)
        wsems = (wsem0, wsem1)
        dst0 = lax.iota(jnp.int32, 16) * DW

        def do_group(h, g):
            def kbody(k, carry):
                tv = idx_v[pl.ds(g * GT + k * 16, 16)]
                src = tv * DW
                dst = dst0 + (h * GW + k * (16 * DW))
                for _ in range(DW):
                    v = plsc.load_gather(tab_v, [src])
                    plsc.store_scatter(stage, [dst], v)
                    src = src + 1
                    dst = dst + 1
                return carry

            lax.fori_loop(0, GT // 16, kbody, 0)
            pltpu.async_copy(
                stage.at[pl.ds(h * GW, GW)],
                out_hbm.at[pl.ds((base + g * GT) * DW, GW)],
                wsems[h])

        def wait_write(h):
            pltpu.make_async_copy(
                out_hbm.at[pl.ds(base * DW, GW)],
                stage.at[pl.ds(h * GW, GW)],
                wsems[h]).wait()

        def outer(i, carry):
            @pl.when(i > 0)
            def _():
                wait_write(0)
            do_group(0, 2 * i)

            @pl.when(i > 0)
            def _():
                wait_write(1)
            do_group(1, 2 * i + 1)
            return carry

        lax.fori_loop(0, nout, outer, 0)
        wait_write(0)
        wait_write(1)

    return gather


def _mlp_body(x_ref, w1_ref, b1_ref, w2_ref, b2_ref, w3_ref, b3_ref, o_ref):
    h1 = jnp.dot(x_ref[:], w1_ref[:], preferred_element_type=jnp.float32)
    h1 = jnp.maximum(h1 + b1_ref[:], 0.0)
    h2 = jnp.dot(h1, w2_ref[:], preferred_element_type=jnp.float32)
    h2 = jnp.maximum(h2 + b2_ref[:], 0.0)
    o_ref[:] = jnp.dot(h2, w3_ref[:], preferred_element_type=jnp.float32) + b3_ref[:]


def _mlp(x, w1, b1, w2, b2, w3, b3, bt):
    b_rows, feat = x.shape
    grid = (b_rows // bt,)
    return pl.pallas_call(
        _mlp_body,
        grid=grid,
        in_specs=[
            pl.BlockSpec((bt, feat), lambda i: (i, 0)),
            pl.BlockSpec(w1.shape, lambda i: (0, 0)),
            pl.BlockSpec(b1.shape, lambda i: (0, 0)),
            pl.BlockSpec(w2.shape, lambda i: (0, 0)),
            pl.BlockSpec(b2.shape, lambda i: (0, 0)),
            pl.BlockSpec(w3.shape, lambda i: (0, 0)),
            pl.BlockSpec(b3.shape, lambda i: (0, 0)),
        ],
        out_specs=pl.BlockSpec((bt, 128), lambda i: (i, 0)),
        out_shape=jax.ShapeDtypeStruct((b_rows, 128), jnp.float32),
    )(x, w1, b1, w2, b2, w3, b3)


def kernel(rna_data, tissue_id, tissue_table, seq_table, W1, b1, W2, b2, W3, b3):
    B, S = rna_data.shape

    tab = jnp.concatenate(
        [seq_table, tissue_table, jnp.zeros((1, D), jnp.float32)], axis=0)
    tab = _renorm_table(tab)

    idx = jnp.concatenate(
        [tissue_id[:, None].astype(jnp.int32) + 65,
         rna_data.astype(jnp.int32),
         jnp.full((B, 1), TAB - 1, jnp.int32)], axis=1)
    idx_flat = idx.reshape(-1)

    tab_bf = tab.astype(jnp.bfloat16)
    tab_i32 = lax.bitcast_convert_type(
        tab_bf.reshape(TAB, D // 2, 2), jnp.int32).reshape(TAB * D // 2)

    rows = B * P
    xw = _make_gather(rows)(tab_i32, idx_flat)
    x2 = lax.bitcast_convert_type(
        xw.reshape(B, P * D // 2), jnp.bfloat16).reshape(B, P * D)

    w1e = jnp.concatenate(
        [W1[:D + SEQ * D], jnp.zeros((D, HID1), jnp.float32)],
        axis=0).astype(jnp.bfloat16)
    w3e = jnp.concatenate(
        [W3, jnp.zeros((HID2, 127), jnp.float32)], axis=1)
    b3e = jnp.concatenate([b3, jnp.zeros((127,), jnp.float32)])[None, :]

    out = _mlp(x2, w1e, b1[None, :], W2, b2[None, :], w3e, b3e, bt=512)
    return out[:, :1]


# speedup vs baseline: 20.2599x; 3.9378x over previous
import functools

import jax
import jax.numpy as jnp
from jax import lax
from jax.experimental import pallas as pl
from jax.experimental.pallas import tpu as pltpu
from jax.experimental.pallas import tpu_sc as plsc

MAX_SEQ_LENGTH = 200
MAX_NORM = 2.0
D = 32
SEQ = 150
P = SEQ + 2
TAB = 96
HID1 = 512
HID2 = 256

NC, NS = 2, 16
NW = NC * NS
CH = 128


def _renorm_body(c_ref, o_ref):
    x = c_ref[:]
    n = jnp.sqrt(jnp.sum(x * x, axis=1, keepdims=True))
    scale = jnp.where(n > MAX_NORM, MAX_NORM / (n + 1e-7), 1.0)
    o_ref[:] = x * scale


def _renorm_table(c):
    return pl.pallas_call(
        _renorm_body,
        out_shape=jax.ShapeDtypeStruct(c.shape, jnp.float32),
    )(c)


def _make_gather(rows):
    DW = 16
    rpw = rows // NW
    nch = rpw // CH
    K = 4
    HALF = K * CH
    nout = nch // K // 2
    mesh = plsc.VectorSubcoreMesh(
        core_axis_name="c", subcore_axis_name="s",
        num_cores=NC, num_subcores=NS)

    @functools.partial(
        pl.kernel,
        mesh=mesh,
        compiler_params=pltpu.CompilerParams(
            use_tc_tiling_on_sc=False, needs_layout_passes=False),
        out_type=jax.ShapeDtypeStruct((rows, DW), jnp.int32),
        scratch_types=[
            pltpu.VMEM((rpw,), jnp.int32),
            pltpu.VMEM_SHARED((TAB, DW), jnp.int32),
            pltpu.VMEM((2 * HALF, DW), jnp.int32),
            pltpu.SemaphoreType.DMA,
            pltpu.SemaphoreType.DMA,
            pltpu.SemaphoreType.DMA,
            pltpu.SemaphoreType.DMA,
        ],
    )
    def gather(tab_hbm, idx_hbm, out_hbm, idx_v, tab_sh, buf,
               gsem0, gsem1, wsem0, wsem1):
        wid = lax.axis_index("s") * NC + lax.axis_index("c")
        base = wid * rpw

        @pl.when(lax.axis_index("s") == 0)
        def _():
            pltpu.sync_copy(tab_hbm, tab_sh)

        plsc.subcore_barrier()
        pltpu.sync_copy(idx_hbm.at[pl.ds(base, rpw)], idx_v)
        gsems = (gsem0, gsem1)
        wsems = (wsem0, wsem1)

        def issue(h, g):
            for b in range(K):
                off = pl.multiple_of(g * HALF + b * CH, CH)
                pltpu.async_copy(
                    tab_sh.at[idx_v.at[pl.ds(off, CH)]],
                    buf.at[pl.ds(h * HALF + b * CH, CH)],
                    gsems[h])

        def wait_half(h, sem):
            pltpu.make_async_copy(
                out_hbm.at[pl.ds(base, HALF)],
                buf.at[pl.ds(h * HALF, HALF)],
                sem).wait()

        def write(h, g):
            pltpu.async_copy(
                buf.at[pl.ds(h * HALF, HALF)],
                out_hbm.at[pl.ds(base + g * HALF, HALF)],
                wsems[h])

        issue(0, 0)

        def outer(i, carry):
            @pl.when(i > 0)
            def _():
                wait_half(1, wsems[1])
            issue(1, 2 * i + 1)
            wait_half(0, gsems[0])
            write(0, 2 * i)

            @pl.when(i + 1 < nout)
            def _():
                wait_half(0, wsems[0])
                issue(0, 2 * i + 2)

            wait_half(1, gsems[1])
            write(1, 2 * i + 1)
            return carry

        lax.fori_loop(0, nout, outer, 0)
        wait_half(0, wsems[0])
        wait_half(1, wsems[1])

    return gather


def _mlp_body(x_ref, w1e_ref, w1o_ref, b1_ref, w2_ref, b2_ref, w3_ref, b3_ref,
              o_ref):
    xi = x_ref[:]
    xe = lax.bitcast_convert_type(xi << 16, jnp.float32).astype(jnp.bfloat16)
    xo = lax.bitcast_convert_type(
        xi & jnp.int32(-65536), jnp.float32).astype(jnp.bfloat16)
    h1 = (jnp.dot(xe, w1e_ref[:], preferred_element_type=jnp.float32)
          + jnp.dot(xo, w1o_ref[:], preferred_element_type=jnp.float32))
    h1 = jnp.maximum(h1 + b1_ref[:], 0.0)
    h2 = jnp.dot(h1, w2_ref[:], preferred_element_type=jnp.float32)
    h2 = jnp.maximum(h2 + b2_ref[:], 0.0)
    o_ref[:] = jnp.dot(h2, w3_ref[:], preferred_element_type=jnp.float32) + b3_ref[:]


def _mlp(x, w1e, w1o, b1, w2, b2, w3, b3, bt):
    b_rows, feat = x.shape
    grid = (b_rows // bt,)
    return pl.pallas_call(
        _mlp_body,
        grid=grid,
        in_specs=[
            pl.BlockSpec((bt, feat), lambda i: (i, 0)),
            pl.BlockSpec(w1e.shape, lambda i: (0, 0)),
            pl.BlockSpec(w1o.shape, lambda i: (0, 0)),
            pl.BlockSpec(b1.shape, lambda i: (0, 0)),
            pl.BlockSpec(w2.shape, lambda i: (0, 0)),
            pl.BlockSpec(b2.shape, lambda i: (0, 0)),
            pl.BlockSpec(w3.shape, lambda i: (0, 0)),
            pl.BlockSpec(b3.shape, lambda i: (0, 0)),
        ],
        out_specs=pl.BlockSpec((bt, 128), lambda i: (i, 0)),
        out_shape=jax.ShapeDtypeStruct((b_rows, 128), jnp.float32),
    )(x, w1e, w1o, b1, w2, b2, w3, b3)


def kernel(rna_data, tissue_id, tissue_table, seq_table, W1, b1, W2, b2, W3, b3):
    B, S = rna_data.shape

    tab = jnp.concatenate(
        [seq_table, tissue_table, jnp.zeros((1, D), jnp.float32)], axis=0)
    tab = _renorm_table(tab)

    idx = jnp.concatenate(
        [tissue_id[:, None].astype(jnp.int32) + 65,
         rna_data.astype(jnp.int32),
         jnp.full((B, 1), TAB - 1, jnp.int32)], axis=1)
    idx_flat = idx.reshape(-1)

    tab_bf = tab.astype(jnp.bfloat16)
    tab_i32 = lax.bitcast_convert_type(
        tab_bf.reshape(TAB, D // 2, 2), jnp.int32)

    rows = B * P
    xw = _make_gather(rows)(tab_i32, idx_flat)
    x2 = xw.reshape(B, P * D // 2)

    w1full = jnp.concatenate(
        [W1[:D + SEQ * D], jnp.zeros((D, HID1), jnp.float32)], axis=0)
    w1p = w1full.reshape(P * D // 2, 2, HID1)
    w1even = w1p[:, 0, :].astype(jnp.bfloat16)
    w1odd = w1p[:, 1, :].astype(jnp.bfloat16)
    w3e = jnp.concatenate(
        [W3, jnp.zeros((HID2, 127), jnp.float32)], axis=1)
    b3e = jnp.concatenate([b3, jnp.zeros((127,), jnp.float32)])[None, :]

    out = _mlp(x2, w1even, w1odd, b1[None, :], W2, b2[None, :], w3e, b3e,
               bt=512)
    return out[:, :1]


# speedup vs baseline: 25.2408x; 1.2459x over previous
import functools

import jax
import jax.numpy as jnp
from jax import lax
from jax.experimental import pallas as pl
from jax.experimental.pallas import tpu as pltpu
from jax.experimental.pallas import tpu_sc as plsc

MAX_SEQ_LENGTH = 200
MAX_NORM = 2.0
D = 32
SEQ = 150
P = SEQ + 2
TAB = 96
HID1 = 512
HID2 = 256

DW2 = 16
NC, NS = 2, 16
NW = NC * NS
CH = 128


def _renorm_body(c_ref, o_ref):
    x = c_ref[:]
    n = jnp.sqrt(jnp.sum(x * x, axis=1, keepdims=True))
    scale = jnp.where(n > MAX_NORM, MAX_NORM / (n + 1e-7), 1.0)
    o_ref[:] = x * scale


def _renorm_table(c):
    return pl.pallas_call(
        _renorm_body,
        out_shape=jax.ShapeDtypeStruct(c.shape, jnp.float32),
    )(c)


def _make_gather(rows):
    DW = 16
    rpw = rows // NW
    nch = rpw // CH
    K = 4
    HALF = K * CH
    nout = nch // K // 2
    mesh = plsc.VectorSubcoreMesh(
        core_axis_name="c", subcore_axis_name="s",
        num_cores=NC, num_subcores=NS)

    @functools.partial(
        pl.kernel,
        mesh=mesh,
        compiler_params=pltpu.CompilerParams(
            use_tc_tiling_on_sc=False, needs_layout_passes=False),
        out_type=jax.ShapeDtypeStruct((rows, DW), jnp.int32),
        scratch_types=[
            pltpu.VMEM((rpw,), jnp.int32),
            pltpu.VMEM_SHARED((TAB, DW), jnp.int32),
            pltpu.VMEM((2 * HALF, DW), jnp.int32),
            pltpu.SemaphoreType.DMA,
            pltpu.SemaphoreType.DMA,
            pltpu.SemaphoreType.DMA,
            pltpu.SemaphoreType.DMA,
        ],
    )
    def gather(tab_hbm, idx_hbm, out_hbm, idx_v, tab_sh, buf,
               gsem0, gsem1, wsem0, wsem1):
        wid = lax.axis_index("s") * NC + lax.axis_index("c")
        base = wid * rpw

        @pl.when(lax.axis_index("s") == 0)
        def _():
            pltpu.sync_copy(tab_hbm, tab_sh)

        plsc.subcore_barrier()
        pltpu.sync_copy(idx_hbm.at[pl.ds(base, rpw)], idx_v)
        gsems = (gsem0, gsem1)
        wsems = (wsem0, wsem1)

        def issue(h, g):
            for b in range(K):
                off = pl.multiple_of(g * HALF + b * CH, CH)
                pltpu.async_copy(
                    tab_sh.at[idx_v.at[pl.ds(off, CH)]],
                    buf.at[pl.ds(h * HALF + b * CH, CH)],
                    gsems[h])

        def wait_half(h, sem):
            pltpu.make_async_copy(
                out_hbm.at[pl.ds(base, HALF)],
                buf.at[pl.ds(h * HALF, HALF)],
                sem).wait()

        def write(h, g):
            pltpu.async_copy(
                buf.at[pl.ds(h * HALF, HALF)],
                out_hbm.at[pl.ds(base + g * HALF, HALF)],
                wsems[h])

        issue(0, 0)

        def outer(i, carry):
            @pl.when(i > 0)
            def _():
                wait_half(1, wsems[1])
            issue(1, 2 * i + 1)
            wait_half(0, gsems[0])
            write(0, 2 * i)

            @pl.when(i + 1 < nout)
            def _():
                wait_half(0, wsems[0])
                issue(0, 2 * i + 2)

            wait_half(1, gsems[1])
            write(1, 2 * i + 1)
            return carry

        lax.fori_loop(0, nout, outer, 0)
        wait_half(0, wsems[0])
        wait_half(1, wsems[1])

    return gather


def _mlp_body(x_ref, w1e_ref, w1o_ref, b1_ref, w2_ref, b2_ref, w3_ref, b3_ref,
              o_ref):
    bt = o_ref.shape[0]
    xi = x_ref[:].reshape(bt, x_ref.shape[0] * 128 // bt)
    xe = lax.bitcast_convert_type(xi << 16, jnp.float32).astype(jnp.bfloat16)
    xo = lax.bitcast_convert_type(
        xi & jnp.int32(-65536), jnp.float32).astype(jnp.bfloat16)
    h1 = (jnp.dot(xe, w1e_ref[:], preferred_element_type=jnp.float32)
          + jnp.dot(xo, w1o_ref[:], preferred_element_type=jnp.float32))
    h1 = jnp.maximum(h1 + b1_ref[:], 0.0)
    h2 = jnp.dot(h1, w2_ref[:], preferred_element_type=jnp.float32)
    h2 = jnp.maximum(h2 + b2_ref[:], 0.0)
    o_ref[:] = jnp.dot(h2, w3_ref[:], preferred_element_type=jnp.float32) + b3_ref[:]


def _mlp(x, w1e, w1o, b1, w2, b2, w3, b3, bt, b_rows):
    xrows = x.shape[0]
    xbt = xrows // (b_rows // bt)
    grid = (b_rows // bt,)
    return pl.pallas_call(
        _mlp_body,
        grid=grid,
        in_specs=[
            pl.BlockSpec((xbt, 128), lambda i: (i, 0)),
            pl.BlockSpec(w1e.shape, lambda i: (0, 0)),
            pl.BlockSpec(w1o.shape, lambda i: (0, 0)),
            pl.BlockSpec(b1.shape, lambda i: (0, 0)),
            pl.BlockSpec(w2.shape, lambda i: (0, 0)),
            pl.BlockSpec(b2.shape, lambda i: (0, 0)),
            pl.BlockSpec(w3.shape, lambda i: (0, 0)),
            pl.BlockSpec(b3.shape, lambda i: (0, 0)),
        ],
        out_specs=pl.BlockSpec((bt, 128), lambda i: (i, 0)),
        out_shape=jax.ShapeDtypeStruct((b_rows, 128), jnp.float32),
    )(x, w1e, w1o, b1, w2, b2, w3, b3)


def kernel(rna_data, tissue_id, tissue_table, seq_table, W1, b1, W2, b2, W3, b3):
    B, S = rna_data.shape

    tab = jnp.concatenate(
        [seq_table, tissue_table, jnp.zeros((1, D), jnp.float32)], axis=0)
    tab = _renorm_table(tab)

    idx = jnp.concatenate(
        [tissue_id[:, None].astype(jnp.int32) + 65,
         rna_data.astype(jnp.int32),
         jnp.full((B, 1), TAB - 1, jnp.int32)], axis=1)
    idx_flat = idx.reshape(-1)

    tab_bf = tab.astype(jnp.bfloat16)
    tab_i32 = lax.bitcast_convert_type(
        tab_bf.reshape(TAB, D // 2, 2), jnp.int32)

    rows = B * P
    xw = _make_gather(rows)(tab_i32, idx_flat)
    xw = xw.reshape(rows * DW2 // 128, 128)

    w1full = jnp.concatenate(
        [W1[:D + SEQ * D], jnp.zeros((D, HID1), jnp.float32)], axis=0)
    w1p = w1full.reshape(P * D // 2, 2, HID1)
    w1even = w1p[:, 0, :].astype(jnp.bfloat16)
    w1odd = w1p[:, 1, :].astype(jnp.bfloat16)
    w3e = jnp.concatenate(
        [W3, jnp.zeros((HID2, 127), jnp.float32)], axis=1)
    b3e = jnp.concatenate([b3, jnp.zeros((127,), jnp.float32)])[None, :]

    out = _mlp(xw, w1even, w1odd, b1[None, :], W2, b2[None, :], w3e, b3e,
               bt=512, b_rows=B)
    return out[:, :1]


# speedup vs baseline: 25.7133x; 1.0187x over previous
import functools

import jax
import jax.numpy as jnp
from jax import lax
from jax.experimental import pallas as pl
from jax.experimental.pallas import tpu as pltpu
from jax.experimental.pallas import tpu_sc as plsc

MAX_SEQ_LENGTH = 200
MAX_NORM = 2.0
D = 32
SEQ = 150
P = SEQ + 2
TAB = 96
HID1 = 512
HID2 = 256

DW2 = 16
NC, NS = 2, 16
NW = NC * NS
CH = 128


def _renorm_body(c_ref, o_ref):
    x = c_ref[:]
    n = jnp.sqrt(jnp.sum(x * x, axis=1, keepdims=True))
    scale = jnp.where(n > MAX_NORM, MAX_NORM / (n + 1e-7), 1.0)
    o_ref[:] = x * scale


def _renorm_table(c):
    return pl.pallas_call(
        _renorm_body,
        out_shape=jax.ShapeDtypeStruct(c.shape, jnp.float32),
    )(c)


def _make_gather(rows):
    DW = 16
    rpw = rows // NW
    nch = rpw // CH
    K = 4
    HALF = K * CH
    nout = nch // K // 2
    mesh = plsc.VectorSubcoreMesh(
        core_axis_name="c", subcore_axis_name="s",
        num_cores=NC, num_subcores=NS)

    @functools.partial(
        pl.kernel,
        mesh=mesh,
        compiler_params=pltpu.CompilerParams(
            use_tc_tiling_on_sc=False, needs_layout_passes=False),
        out_type=jax.ShapeDtypeStruct((rows, DW), jnp.int32),
        scratch_types=[
            pltpu.VMEM((rpw,), jnp.int32),
            pltpu.VMEM_SHARED((TAB, DW), jnp.int32),
            pltpu.VMEM((2 * HALF, DW), jnp.int32),
            pltpu.SemaphoreType.DMA,
            pltpu.SemaphoreType.DMA,
            pltpu.SemaphoreType.DMA,
            pltpu.SemaphoreType.DMA,
        ],
    )
    def gather(tab_hbm, idx_hbm, out_hbm, idx_v, tab_sh, buf,
               gsem0, gsem1, wsem0, wsem1):
        wid = lax.axis_index("s") * NC + lax.axis_index("c")
        base = wid * rpw

        @pl.when(lax.axis_index("s") == 0)
        def _():
            pltpu.sync_copy(tab_hbm, tab_sh)

        plsc.subcore_barrier()
        pltpu.sync_copy(idx_hbm.at[pl.ds(base, rpw)], idx_v)
        gsems = (gsem0, gsem1)
        wsems = (wsem0, wsem1)

        def issue(h, g):
            for b in range(K):
                off = pl.multiple_of(g * HALF + b * CH, CH)
                pltpu.async_copy(
                    tab_sh.at[idx_v.at[pl.ds(off, CH)]],
                    buf.at[pl.ds(h * HALF + b * CH, CH)],
                    gsems[h])

        def wait_half(h, sem):
            pltpu.make_async_copy(
                out_hbm.at[pl.ds(base, HALF)],
                buf.at[pl.ds(h * HALF, HALF)],
                sem).wait()

        def write(h, g):
            pltpu.async_copy(
                buf.at[pl.ds(h * HALF, HALF)],
                out_hbm.at[pl.ds(base + g * HALF, HALF)],
                wsems[h])

        issue(0, 0)

        def outer(i, carry):
            @pl.when(i > 0)
            def _():
                wait_half(1, wsems[1])
            issue(1, 2 * i + 1)
            wait_half(0, gsems[0])
            write(0, 2 * i)

            @pl.when(i + 1 < nout)
            def _():
                wait_half(0, wsems[0])
                issue(0, 2 * i + 2)

            wait_half(1, gsems[1])
            write(1, 2 * i + 1)
            return carry

        lax.fori_loop(0, nout, outer, 0)
        wait_half(0, wsems[0])
        wait_half(1, wsems[1])

    return gather


NW1 = D + SEQ * D
NH = NW1 // 2
NWORD = P * D // 2


def _mlp_body(x_ref, w1_ref, b1_ref, w2_ref, b2_ref, w3_ref, b3_ref,
              o_ref, w1e_s, w1o_s):
    @pl.when(pl.program_id(0) == 0)
    def _():
        w13 = w1_ref[:].reshape(NH, 2, HID1)
        w1e_s[pl.ds(0, NH), :] = w13[:, 0, :].astype(jnp.bfloat16)
        w1o_s[pl.ds(0, NH), :] = w13[:, 1, :].astype(jnp.bfloat16)
        zpad = jnp.zeros((NWORD - NH, HID1), jnp.bfloat16)
        w1e_s[pl.ds(NH, NWORD - NH), :] = zpad
        w1o_s[pl.ds(NH, NWORD - NH), :] = zpad

    bt = o_ref.shape[0]
    xi = x_ref[:].reshape(bt, NWORD)
    xe = lax.bitcast_convert_type(xi << 16, jnp.float32).astype(jnp.bfloat16)
    xo = lax.bitcast_convert_type(
        xi & jnp.int32(-65536), jnp.float32).astype(jnp.bfloat16)
    h1 = (jnp.dot(xe, w1e_s[:], preferred_element_type=jnp.float32)
          + jnp.dot(xo, w1o_s[:], preferred_element_type=jnp.float32))
    h1 = jnp.maximum(h1 + b1_ref[:], 0.0)
    h2 = jnp.dot(h1, w2_ref[:], preferred_element_type=jnp.float32)
    h2 = jnp.maximum(h2 + b2_ref[:], 0.0)
    o_ref[:] = jnp.dot(h2, w3_ref[:], preferred_element_type=jnp.float32) + b3_ref[:]


def _mlp(x, w1, b1, w2, b2, w3, b3, bt, b_rows):
    xrows = x.shape[0]
    xbt = xrows // (b_rows // bt)
    grid = (b_rows // bt,)
    return pl.pallas_call(
        _mlp_body,
        grid=grid,
        in_specs=[
            pl.BlockSpec((xbt, 128), lambda i: (i, 0)),
            pl.BlockSpec((NW1, HID1), lambda i: (0, 0)),
            pl.BlockSpec(b1.shape, lambda i: (0, 0)),
            pl.BlockSpec(w2.shape, lambda i: (0, 0)),
            pl.BlockSpec(b2.shape, lambda i: (0, 0)),
            pl.BlockSpec(w3.shape, lambda i: (0, 0)),
            pl.BlockSpec(b3.shape, lambda i: (0, 0)),
        ],
        out_specs=pl.BlockSpec((bt, 128), lambda i: (i, 0)),
        out_shape=jax.ShapeDtypeStruct((b_rows, 128), jnp.float32),
        scratch_shapes=[
            pltpu.VMEM((NWORD, HID1), jnp.bfloat16),
            pltpu.VMEM((NWORD, HID1), jnp.bfloat16),
        ],
    )(x, w1, b1, w2, b2, w3, b3)


def kernel(rna_data, tissue_id, tissue_table, seq_table, W1, b1, W2, b2, W3, b3):
    B, S = rna_data.shape

    tab = jnp.concatenate(
        [seq_table, tissue_table, jnp.zeros((1, D), jnp.float32)], axis=0)
    tab = _renorm_table(tab)

    idx = jnp.concatenate(
        [tissue_id[:, None].astype(jnp.int32) + 65,
         rna_data.astype(jnp.int32),
         jnp.full((B, 1), TAB - 1, jnp.int32)], axis=1)
    idx_flat = idx.reshape(-1)

    tab_bf = tab.astype(jnp.bfloat16)
    tab_i32 = lax.bitcast_convert_type(
        tab_bf.reshape(TAB, D // 2, 2), jnp.int32)

    rows = B * P
    xw = _make_gather(rows)(tab_i32, idx_flat)
    xw = xw.reshape(rows * DW2 // 128, 128)

    w3e = jnp.concatenate(
        [W3, jnp.zeros((HID2, 127), jnp.float32)], axis=1)
    b3e = jnp.concatenate([b3, jnp.zeros((127,), jnp.float32)])[None, :]

    out = _mlp(xw, W1, b1[None, :], W2, b2[None, :], w3e, b3e,
               bt=512, b_rows=B)
    return out[:, :1]


# speedup vs baseline: 28.5397x; 1.1099x over previous
import functools

import jax
import jax.numpy as jnp
from jax import lax
from jax.experimental import pallas as pl
from jax.experimental.pallas import tpu as pltpu
from jax.experimental.pallas import tpu_sc as plsc

MAX_SEQ_LENGTH = 200
MAX_NORM = 2.0
D = 32
SEQ = 150
P = SEQ + 2
TAB = 96
HID1 = 512
HID2 = 256

DW2 = 16
NC, NS = 2, 16
NW = NC * NS
CH = 128


def _renorm_body(c_ref, o_ref):
    x = c_ref[:]
    n = jnp.sqrt(jnp.sum(x * x, axis=1, keepdims=True))
    scale = jnp.where(n > MAX_NORM, MAX_NORM / (n + 1e-7), 1.0)
    o_ref[:] = x * scale


def _renorm_table(c):
    return pl.pallas_call(
        _renorm_body,
        out_shape=jax.ShapeDtypeStruct(c.shape, jnp.float32),
    )(c)


def _make_gather(rows):
    DW = 16
    rpw = rows // NW
    nch = rpw // CH
    K = 4
    HALF = K * CH
    nout = nch // K // 2
    mesh = plsc.VectorSubcoreMesh(
        core_axis_name="c", subcore_axis_name="s",
        num_cores=NC, num_subcores=NS)

    @functools.partial(
        pl.kernel,
        mesh=mesh,
        compiler_params=pltpu.CompilerParams(
            use_tc_tiling_on_sc=False, needs_layout_passes=False),
        out_type=jax.ShapeDtypeStruct((rows, DW), jnp.int32),
        scratch_types=[
            pltpu.VMEM((rpw,), jnp.int32),
            pltpu.VMEM_SHARED((TAB, DW), jnp.int32),
            pltpu.VMEM((2 * HALF, DW), jnp.int32),
            pltpu.SemaphoreType.DMA,
            pltpu.SemaphoreType.DMA,
            pltpu.SemaphoreType.DMA,
            pltpu.SemaphoreType.DMA,
        ],
    )
    def gather(tab_hbm, idx_hbm, out_hbm, idx_v, tab_sh, buf,
               gsem0, gsem1, wsem0, wsem1):
        wid = lax.axis_index("s") * NC + lax.axis_index("c")
        base = wid * rpw

        @pl.when(lax.axis_index("s") == 0)
        def _():
            pltpu.sync_copy(tab_hbm, tab_sh)

        plsc.subcore_barrier()
        pltpu.sync_copy(idx_hbm.at[pl.ds(base, rpw)], idx_v)
        gsems = (gsem0, gsem1)
        wsems = (wsem0, wsem1)

        def issue(h, g):
            for b in range(K):
                off = pl.multiple_of(g * HALF + b * CH, CH)
                pltpu.async_copy(
                    tab_sh.at[idx_v.at[pl.ds(off, CH)]],
                    buf.at[pl.ds(h * HALF + b * CH, CH)],
                    gsems[h])

        def wait_half(h, sem):
            pltpu.make_async_copy(
                out_hbm.at[pl.ds(base, HALF)],
                buf.at[pl.ds(h * HALF, HALF)],
                sem).wait()

        def write(h, g):
            pltpu.async_copy(
                buf.at[pl.ds(h * HALF, HALF)],
                out_hbm.at[pl.ds(base + g * HALF, HALF)],
                wsems[h])

        issue(0, 0)

        def outer(i, carry):
            @pl.when(i > 0)
            def _():
                wait_half(1, wsems[1])
            issue(1, 2 * i + 1)
            wait_half(0, gsems[0])
            write(0, 2 * i)

            @pl.when(i + 1 < nout)
            def _():
                wait_half(0, wsems[0])
                issue(0, 2 * i + 2)

            wait_half(1, gsems[1])
            write(1, 2 * i + 1)
            return carry

        lax.fori_loop(0, nout, outer, 0)
        wait_half(0, wsems[0])
        wait_half(1, wsems[1])

    return gather


NW1 = D + SEQ * D
NH = NW1 // 2
NWORD = P * D // 2


def _mlp_body(x_ref, w1_ref, b1_ref, w2_ref, b2_ref, w3_ref, b3_ref,
              o_ref, w1e_s, w1o_s):
    @pl.when(pl.program_id(0) == 0)
    def _():
        w13 = w1_ref[:].reshape(NW1 // D, 2, D // 2, HID1)
        w1e_s[pl.ds(0, NH), :] = w13[:, 0, :, :].reshape(NH, HID1).astype(jnp.bfloat16)
        w1o_s[pl.ds(0, NH), :] = w13[:, 1, :, :].reshape(NH, HID1).astype(jnp.bfloat16)
        zpad = jnp.zeros((NWORD - NH, HID1), jnp.bfloat16)
        w1e_s[pl.ds(NH, NWORD - NH), :] = zpad
        w1o_s[pl.ds(NH, NWORD - NH), :] = zpad

    bt = o_ref.shape[0]
    xi = x_ref[:].reshape(bt, NWORD)
    xe = lax.bitcast_convert_type(xi << 16, jnp.float32).astype(jnp.bfloat16)
    xo = lax.bitcast_convert_type(
        xi & jnp.int32(-65536), jnp.float32).astype(jnp.bfloat16)
    h1 = (jnp.dot(xe, w1e_s[:], preferred_element_type=jnp.float32)
          + jnp.dot(xo, w1o_s[:], preferred_element_type=jnp.float32))
    h1 = jnp.maximum(h1 + b1_ref[:], 0.0)
    h2 = jnp.dot(h1, w2_ref[:], preferred_element_type=jnp.float32)
    h2 = jnp.maximum(h2 + b2_ref[:], 0.0)
    o_ref[:] = jnp.dot(h2, w3_ref[:], preferred_element_type=jnp.float32) + b3_ref[:]


def _mlp(x, w1, b1, w2, b2, w3, b3, bt, b_rows):
    xrows = x.shape[0]
    xbt = xrows // (b_rows // bt)
    grid = (b_rows // bt,)
    return pl.pallas_call(
        _mlp_body,
        grid=grid,
        in_specs=[
            pl.BlockSpec((xbt, 128), lambda i: (i, 0)),
            pl.BlockSpec((NW1, HID1), lambda i: (0, 0)),
            pl.BlockSpec(b1.shape, lambda i: (0, 0)),
            pl.BlockSpec(w2.shape, lambda i: (0, 0)),
            pl.BlockSpec(b2.shape, lambda i: (0, 0)),
            pl.BlockSpec(w3.shape, lambda i: (0, 0)),
            pl.BlockSpec(b3.shape, lambda i: (0, 0)),
        ],
        out_specs=pl.BlockSpec((bt, 128), lambda i: (i, 0)),
        out_shape=jax.ShapeDtypeStruct((b_rows, 128), jnp.float32),
        scratch_shapes=[
            pltpu.VMEM((NWORD, HID1), jnp.bfloat16),
            pltpu.VMEM((NWORD, HID1), jnp.bfloat16),
        ],
    )(x, w1, b1, w2, b2, w3, b3)


def kernel(rna_data, tissue_id, tissue_table, seq_table, W1, b1, W2, b2, W3, b3):
    B, S = rna_data.shape

    tab = jnp.concatenate(
        [seq_table, tissue_table, jnp.zeros((1, D), jnp.float32)], axis=0)
    tab = _renorm_table(tab)

    idx = jnp.concatenate(
        [tissue_id[:, None].astype(jnp.int32) + 65,
         rna_data.astype(jnp.int32),
         jnp.full((B, 1), TAB - 1, jnp.int32)], axis=1)
    idx_flat = idx.reshape(-1)

    tab_bf = tab.astype(jnp.bfloat16)
    tab_i32 = lax.bitcast_convert_type(
        tab_bf.reshape(TAB, 2, D // 2).transpose(0, 2, 1), jnp.int32)

    rows = B * P
    xw = _make_gather(rows)(tab_i32, idx_flat)
    xw = xw.reshape(rows * DW2 // 128, 128)

    w3e = jnp.concatenate(
        [W3, jnp.zeros((HID2, 127), jnp.float32)], axis=1)
    b3e = jnp.concatenate([b3, jnp.zeros((127,), jnp.float32)])[None, :]

    out = _mlp(xw, W1, b1[None, :], W2, b2[None, :], w3e, b3e,
               bt=512, b_rows=B)
    return out[:, :1]
